# baseline (speedup 1.0000x reference)
import functools

import jax
import jax.numpy as jnp
from jax import lax
from jax.experimental import pallas as pl
from jax.experimental.pallas import tpu as pltpu
from jax.experimental.pallas import tpu_sc as plsc

BATCH = 16384
HIST = 50
DIM = 64
NUM_CORES = 2
NUM_SUBCORES = 16
NUM_WORKERS = NUM_CORES * NUM_SUBCORES
ROWS_PER_W = BATCH // NUM_WORKERS
CHUNK = 128
NCHUNK = ROWS_PER_W // CHUNK
LANES = 16
NGROUP = ROWS_PER_W // LANES


def _sc_body(table_hbm, idx_hbm, out_hbm, slab_v, idx_t, acc_v, sem):
    wid = lax.axis_index("s") * NUM_CORES + lax.axis_index("c")
    pltpu.sync_copy(idx_hbm.at[wid], slab_v)

    biota = lax.iota(jnp.int32, LANES) * HIST

    def transpose_row(l):
        for g in range(NGROUP):
            addrs = biota + (g * (LANES * HIST) + l)
            v = plsc.load_gather(slab_v, [addrs])
            s, o = divmod(g * LANES, CHUNK)
            idx_t[l, s, pl.ds(o, LANES)] = v

    transpose_row(0)
    for s in range(NCHUNK):
        pltpu.async_copy(
            table_hbm.at[idx_t.at[0, s]],
            acc_v.at[pl.ds(s * CHUNK, CHUNK)],
            sem,
        )
    for s in range(NCHUNK):
        pltpu.make_async_copy(
            table_hbm.at[idx_t.at[0, s]],
            acc_v.at[pl.ds(s * CHUNK, CHUNK)],
            sem,
        ).wait()

    @pl.loop(1, HIST)
    def _(l):
        transpose_row(l)
        for s in range(NCHUNK):
            pltpu.async_copy(
                table_hbm.at[idx_t.at[l, s]],
                acc_v.at[pl.ds(s * CHUNK, CHUNK)],
                sem,
                add=True,
            )

    @pl.loop(1, HIST)
    def _(l):
        for s in range(NCHUNK):
            pltpu.make_async_copy(
                table_hbm.at[idx_t.at[l, s]],
                acc_v.at[pl.ds(s * CHUNK, CHUNK)],
                sem,
            ).wait()

    pltpu.sync_copy(acc_v, out_hbm.at[pl.ds(wid * ROWS_PER_W, ROWS_PER_W)])


@functools.partial(
    pl.kernel,
    out_type=jax.ShapeDtypeStruct((BATCH, DIM), jnp.float32),
    mesh=plsc.VectorSubcoreMesh(
        core_axis_name="c", subcore_axis_name="s",
        num_cores=NUM_CORES, num_subcores=NUM_SUBCORES,
    ),
    scratch_types=[
        pltpu.VMEM((ROWS_PER_W * HIST,), jnp.int32),
        pltpu.VMEM((HIST, NCHUNK, CHUNK), jnp.int32),
        pltpu.VMEM((ROWS_PER_W, DIM), jnp.float32),
        pltpu.SemaphoreType.DMA,
    ],
    compiler_params=pltpu.CompilerParams(
        use_tc_tiling_on_sc=False, needs_layout_passes=False,
    ),
)
def _sc_embed_sum(table_hbm, idx_hbm, out_hbm, slab_v, idx_t, acc_v, sem):
    _sc_body(table_hbm, idx_hbm, out_hbm, slab_v, idx_t, acc_v, sem)


def kernel(x, table):
    idx = x.reshape(NUM_WORKERS, ROWS_PER_W * HIST)
    return _sc_embed_sum(table, idx)
